# baseline (speedup 1.0000x reference)
import functools

import jax
import jax.numpy as jnp
from jax import lax
from jax.experimental import pallas as pl
from jax.experimental.pallas import tpu as pltpu
from jax.experimental.pallas import tpu_sc as plsc

NN = 10000
EE = 320000
DH = 128
DO = 64
L = 16
NC = 2
NS = 16
NW = NC * NS
EPW = EE // NW
C = 80
NCHUNK = EPW // C
RPT = 624
RPT_LAST = NN - RPT * (NS - 1)
ZR = 16


def _edge_pass(D):
    mesh = plsc.VectorSubcoreMesh(core_axis_name="c", subcore_axis_name="s")

    @functools.partial(
        pl.kernel,
        out_type=[
            jax.ShapeDtypeStruct((NC, NN, D), jnp.float32),
            jax.ShapeDtypeStruct((NW, 1, NN), jnp.float32),
        ],
        mesh=mesh,
        scratch_types=[
            pltpu.VMEM((C,), jnp.int32),
            pltpu.VMEM((C,), jnp.int32),
            pltpu.VMEM((C,), jnp.int32),
            pltpu.VMEM((C,), jnp.int32),
            pltpu.VMEM((C,), jnp.int32),
            pltpu.VMEM((C,), jnp.int32),
            pltpu.VMEM((C,), jnp.int32),
            pltpu.VMEM((C,), jnp.int32),
            pltpu.VMEM((C,), jnp.float32),
            pltpu.VMEM((C, D), jnp.float32),
            pltpu.VMEM((C, D), jnp.float32),
            pltpu.VMEM((NN,), jnp.float32),
            pltpu.VMEM((NN,), jnp.float32),
            pltpu.VMEM((NN,), jnp.float32),
            pltpu.VMEM((ZR, D), jnp.float32),
            pltpu.VMEM_SHARED((NN, D), jnp.float32),
        ] + [pltpu.SemaphoreType.DMA] * 9,
        compiler_params=pltpu.CompilerParams(
            needs_layout_passes=False, use_tc_tiling_on_sc=False),
    )
    def k(h_hbm, asn_hbm, adn_hbm, src_hbm, dst_hbm,
          acc_out, s_out,
          sr0, sr1, sr2, sr3, dr0, dr1, dr2, dr3,
          w_buf, rows0, rows1, as_tab, ad_tab, s_tile, zbuf,
          acc, si0, si1, si2, si3, sg0, sg1, ss0, ss1, sz):
        srcs = (sr0, sr1, sr2, sr3)
        dsts = (dr0, dr1, dr2, dr3)
        cid = lax.axis_index("c")
        sid = lax.axis_index("s")
        wid = cid * NS + sid

        sem_i = (si0, si1, si2, si3)
        sem_g = (sg0, sg1)
        sem_s = (ss0, ss1)
        rows_b = (rows0, rows1)

        zero16 = jnp.zeros((L,), jnp.float32)

        tab_a = pltpu.async_copy(asn_hbm, as_tab, sg0)
        tab_b = pltpu.async_copy(adn_hbm, ad_tab, sg1)

        @pl.loop(0, ZR)
        def _(r):
            for j in range(D // L):
                zbuf[r, pl.ds(j * L, L)] = zero16

        @pl.loop(0, NN // L)
        def _(i):
            s_tile[pl.ds(pl.multiple_of(i * L, L), L)] = zero16

        row_start = pl.multiple_of(sid * RPT, 8)
        nz = RPT // ZR

        def _zslice(z):
            return acc.at[pl.ds(pl.multiple_of(row_start + z * ZR, ZR), ZR)]

        @pl.loop(0, nz)
        def _(z):
            pltpu.sync_copy(zbuf, _zslice(z))

        @pl.when(sid == NS - 1)
        def _():
            @pl.loop(nz, RPT_LAST // ZR)
            def _(z):
                pltpu.sync_copy(zbuf, _zslice(z))

        tab_a.wait()
        tab_b.wait()
        plsc.subcore_barrier()

        def _ibase(c):
            return pl.multiple_of(wid * EPW + c * C, 8)

        def issue_idx(c, slot):
            base = _ibase(c)
            pltpu.async_copy(src_hbm.at[pl.ds(base, C)], srcs[slot],
                             sem_i[slot])
            pltpu.async_copy(dst_hbm.at[pl.ds(base, C)], dsts[slot],
                             sem_i[slot])

        def wait_idx(c, slot):
            base = _ibase(c)
            pltpu.make_async_copy(src_hbm.at[pl.ds(base, C)], srcs[slot],
                                  sem_i[slot]).wait()
            pltpu.make_async_copy(dst_hbm.at[pl.ds(base, C)], dsts[slot],
                                  sem_i[slot]).wait()

        def wait_scatter(rb, slot):
            pltpu.make_async_copy(rows_b[rb], acc.at[dsts[slot]],
                                  sem_s[rb]).wait()

        def chunk_body(c, j, steady):
            rb = j % 2
            rows = rows_b[rb]
            wait_idx(c, j)
            if steady:
                wait_scatter(rb, (j + 2) % 4)
            else:
                @pl.when(c >= 2)
                def _():
                    wait_scatter(rb, (j + 2) % 4)
            @pl.when(c + 2 < NCHUNK)
            def _():
                issue_idx(c + 2, (j + 2) % 4)
            gat = pltpu.async_copy(h_hbm.at[srcs[j]], rows, sem_g[rb])

            @pl.loop(0, C // L)
            def _(g):
                off = pl.multiple_of(g * L, L)
                s16 = srcs[j][pl.ds(off, L)]
                d16 = dsts[j][pl.ds(off, L)]
                e = plsc.load_gather(as_tab, [s16]) + plsc.load_gather(ad_tab, [d16])
                e = jnp.maximum(e, 0.2 * e)
                w = jnp.exp(e)
                w_buf[pl.ds(off, L)] = w
                plsc.addupdate_scatter(s_tile, [d16], w)

            gat.wait()

            @pl.loop(0, C // L)
            def _(g):
                off = pl.multiple_of(g * L, L)
                w16 = w_buf[pl.ds(off, L)]
                for jj in range(L):
                    wj = jnp.full((L,), w16[jj])
                    for kk in range(D // L):
                        rows[off + jj, pl.ds(kk * L, L)] = (
                            rows[off + jj, pl.ds(kk * L, L)] * wj)

            pltpu.async_copy(rows, acc.at[dsts[j]], sem_s[rb], add=True)

        issue_idx(0, 0)
        issue_idx(1, 1)

        @pl.loop(0, 1)
        def _(q):
            for j in range(2):
                chunk_body(q * 4 + j, j, steady=False)
            for j in range(2, 4):
                chunk_body(q * 4 + j, j, steady=True)

        @pl.loop(1, (NCHUNK - 1) // 4)
        def _(q):
            for j in range(4):
                chunk_body(q * 4 + j, j, steady=True)

        chunk_body(NCHUNK - 1, 0, steady=True)

        wait_scatter(1, 3)
        wait_scatter(0, 0)

        plsc.subcore_barrier()

        @pl.when(sid < NS - 1)
        def _():
            pltpu.sync_copy(acc.at[pl.ds(row_start, RPT)],
                            acc_out.at[cid, pl.ds(row_start, RPT)])

        @pl.when(sid == NS - 1)
        def _():
            pltpu.sync_copy(acc.at[pl.ds(row_start, RPT_LAST)],
                            acc_out.at[cid, pl.ds(row_start, RPT_LAST)])

        pltpu.sync_copy(s_tile, s_out.at[wid, 0])

    return k


def _dense1_body(x_ref, W_ref, as_ref, ad_ref,
                 ha_ref, hb_ref, asn_ref, adn_ref, lw_ref):
    h = jnp.dot(x_ref[...], W_ref[...], preferred_element_type=jnp.float32)
    ha_ref[...] = h[:, :DO]
    hb_ref[...] = h[:, DO:]
    asn = jnp.sum(h * as_ref[...], axis=1)
    adn = jnp.sum(h * ad_ref[...], axis=1)
    asn_ref[...] = asn
    adn_ref[...] = adn
    e = asn + adn
    lw_ref[...] = jnp.exp(jnp.maximum(e, 0.2 * e))


def _combine2_body(accA_ref, accB_ref, sp_ref, ha_ref, hb_ref, lw_ref, b_ref,
                   W_ref, as_ref, ad_ref, h2_ref, asn_ref, adn_ref, lw2_ref):
    lw = lw_ref[...]
    s = jnp.sum(sp_ref[...][:, 0, :], axis=0) + lw
    inv = (1.0 / (s + 1e-16))[:, None]
    b = b_ref[...]
    oa = (accA_ref[0] + accA_ref[1] + lw[:, None] * ha_ref[...]) * inv + b[:, :DO]
    ob = (accB_ref[0] + accB_ref[1] + lw[:, None] * hb_ref[...]) * inv + b[:, DO:]
    oa = jnp.maximum(oa, 0.0)
    ob = jnp.maximum(ob, 0.0)
    W = W_ref[...]
    h2 = (jnp.dot(oa, W[:DO, :], preferred_element_type=jnp.float32)
          + jnp.dot(ob, W[DO:, :], preferred_element_type=jnp.float32))
    h2_ref[...] = h2
    asn = jnp.sum(h2 * as_ref[...], axis=1)
    adn = jnp.sum(h2 * ad_ref[...], axis=1)
    asn_ref[...] = asn
    adn_ref[...] = adn
    e2 = asn + adn
    lw2_ref[...] = jnp.exp(jnp.maximum(e2, 0.2 * e2))


def _final_body(acc_ref, sp_ref, h_ref, lw_ref, b_ref, out_ref):
    lw = lw_ref[...]
    acc = acc_ref[0] + acc_ref[1] + lw[:, None] * h_ref[...]
    s = jnp.sum(sp_ref[...][:, 0, :], axis=0) + lw
    o = acc / (s + 1e-16)[:, None] + b_ref[...]
    m = jnp.max(o, axis=1, keepdims=True)
    z = o - m
    out_ref[...] = z - jnp.log(jnp.sum(jnp.exp(z), axis=1, keepdims=True))


@functools.lru_cache(maxsize=1)
def _edge64():
    return _edge_pass(DO)


def kernel(x, edge_index, new_edge_indexs, W1, a_src1, a_dst1, b1,
           W2, a_src2, a_dst2, b2):
    f32 = jnp.float32
    src = edge_index[0]
    dst = edge_index[1]
    ep = _edge64()

    ha, hb, asn1, adn1, lw1 = pl.pallas_call(
        _dense1_body,
        out_shape=[
            jax.ShapeDtypeStruct((NN, DO), f32),
            jax.ShapeDtypeStruct((NN, DO), f32),
            jax.ShapeDtypeStruct((NN,), f32),
            jax.ShapeDtypeStruct((NN,), f32),
            jax.ShapeDtypeStruct((NN,), f32),
        ],
    )(x, W1, a_src1.reshape(1, -1), a_dst1.reshape(1, -1))

    accA, sA = ep(ha, asn1, adn1, src, dst)
    accB, _sB = ep(hb, asn1, adn1, src, dst)

    h2, asn2, adn2, lw2 = pl.pallas_call(
        _combine2_body,
        out_shape=[
            jax.ShapeDtypeStruct((NN, DO), f32),
            jax.ShapeDtypeStruct((NN,), f32),
            jax.ShapeDtypeStruct((NN,), f32),
            jax.ShapeDtypeStruct((NN,), f32),
        ],
    )(accA, accB, sA, ha, hb, lw1, b1.reshape(1, -1), W2,
      a_src2.reshape(1, -1), a_dst2.reshape(1, -1))

    acc2, s2 = ep(h2, asn2, adn2, src, dst)

    out = pl.pallas_call(
        _final_body,
        out_shape=jax.ShapeDtypeStruct((NN, DO), f32),
    )(acc2, s2, h2, lw2, b2.reshape(1, -1))
    return out


# speedup vs baseline: 1.9015x; 1.9015x over previous
import functools

import jax
import jax.numpy as jnp
from jax import lax
from jax.experimental import pallas as pl
from jax.experimental.pallas import tpu as pltpu
from jax.experimental.pallas import tpu_sc as plsc

NN = 10000
EE = 320000
DH = 128
DO = 64
L = 16
NC = 2
NS = 16
NW = NC * NS
EPW = EE // NW
C = 80
NCHUNK = EPW // C
RPT = 624
RPT_LAST = NN - RPT * (NS - 1)
ZR = 16


def _edge_pass(D):
    mesh = plsc.VectorSubcoreMesh(core_axis_name="c", subcore_axis_name="s")

    @functools.partial(
        pl.kernel,
        out_type=[
            jax.ShapeDtypeStruct((NC, NN, D), jnp.float32),
            jax.ShapeDtypeStruct((NW, 1, NN), jnp.float32),
        ],
        mesh=mesh,
        scratch_types=[
            pltpu.VMEM((C,), jnp.int32),
            pltpu.VMEM((C,), jnp.int32),
            pltpu.VMEM((C,), jnp.int32),
            pltpu.VMEM((C,), jnp.int32),
            pltpu.VMEM((C,), jnp.int32),
            pltpu.VMEM((C,), jnp.int32),
            pltpu.VMEM((C,), jnp.int32),
            pltpu.VMEM((C,), jnp.int32),
            pltpu.VMEM((C,), jnp.float32),
            pltpu.VMEM((C,), jnp.float32),
            pltpu.VMEM((C, D), jnp.float32),
            pltpu.VMEM((C, D), jnp.float32),
            pltpu.VMEM((NN,), jnp.float32),
            pltpu.VMEM((NN,), jnp.float32),
            pltpu.VMEM((NN,), jnp.float32),
            pltpu.VMEM((ZR, D), jnp.float32),
            pltpu.VMEM_SHARED((NN, D), jnp.float32),
        ] + [pltpu.SemaphoreType.DMA] * 9,
        compiler_params=pltpu.CompilerParams(
            needs_layout_passes=False, use_tc_tiling_on_sc=False),
    )
    def k(h_hbm, asn_hbm, adn_hbm, src_hbm, dst_hbm,
          acc_out, s_out,
          sr0, sr1, sr2, sr3, dr0, dr1, dr2, dr3,
          w0, w1, rows0, rows1, as_tab, ad_tab, s_tile, zbuf,
          acc, si0, si1, si2, si3, sg0, sg1, ss0, ss1, sz):
        srcs = (sr0, sr1, sr2, sr3)
        dsts = (dr0, dr1, dr2, dr3)
        cid = lax.axis_index("c")
        sid = lax.axis_index("s")
        wid = cid * NS + sid

        sem_i = (si0, si1, si2, si3)
        sem_g = (sg0, sg1)
        sem_s = (ss0, ss1)
        rows_b = (rows0, rows1)
        w_bufs = (w0, w1)

        zero16 = jnp.zeros((L,), jnp.float32)

        tab_a = pltpu.async_copy(asn_hbm, as_tab, sg0)
        tab_b = pltpu.async_copy(adn_hbm, ad_tab, sg1)

        @pl.loop(0, ZR)
        def _(r):
            for j in range(D // L):
                zbuf[r, pl.ds(j * L, L)] = zero16

        @pl.loop(0, NN // L)
        def _(i):
            s_tile[pl.ds(pl.multiple_of(i * L, L), L)] = zero16

        row_start = pl.multiple_of(sid * RPT, 8)
        nz = RPT // ZR

        def _zslice(z):
            return acc.at[pl.ds(pl.multiple_of(row_start + z * ZR, ZR), ZR)]

        @pl.loop(0, nz)
        def _(z):
            pltpu.sync_copy(zbuf, _zslice(z))

        @pl.when(sid == NS - 1)
        def _():
            @pl.loop(nz, RPT_LAST // ZR)
            def _(z):
                pltpu.sync_copy(zbuf, _zslice(z))

        tab_a.wait()
        tab_b.wait()
        plsc.subcore_barrier()

        def _ibase(c):
            return pl.multiple_of(wid * EPW + c * C, 8)

        def issue_idx(c, slot):
            base = _ibase(c)
            pltpu.async_copy(src_hbm.at[pl.ds(base, C)], srcs[slot],
                             sem_i[slot])
            pltpu.async_copy(dst_hbm.at[pl.ds(base, C)], dsts[slot],
                             sem_i[slot])

        def wait_idx(c, slot):
            base = _ibase(c)
            pltpu.make_async_copy(src_hbm.at[pl.ds(base, C)], srcs[slot],
                                  sem_i[slot]).wait()
            pltpu.make_async_copy(dst_hbm.at[pl.ds(base, C)], dsts[slot],
                                  sem_i[slot]).wait()

        def wait_scatter(rb, slot):
            pltpu.make_async_copy(rows_b[rb], acc.at[dsts[slot]],
                                  sem_s[rb]).wait()

        def compute_w(j, wb):
            for g in range(C // L):
                off = g * L
                s16 = srcs[j][pl.ds(off, L)]
                d16 = dsts[j][pl.ds(off, L)]
                e = plsc.load_gather(as_tab, [s16]) + plsc.load_gather(ad_tab, [d16])
                e = jnp.maximum(e, 0.2 * e)
                w = jnp.exp(e)
                w_bufs[wb][pl.ds(off, L)] = w
                plsc.addupdate_scatter(s_tile, [d16], w)

        def scale_rows(rb):
            rows = rows_b[rb]
            for g in range(C // L):
                off = g * L
                w16 = w_bufs[rb][pl.ds(off, L)]
                for jj in range(L):
                    wj = jnp.full((L,), w16[jj])
                    for kk in range(D // L):
                        rows[off + jj, pl.ds(kk * L, L)] = (
                            rows[off + jj, pl.ds(kk * L, L)] * wj)

        def issue_gather(j, rb):
            pltpu.async_copy(h_hbm.at[srcs[j]], rows_b[rb], sem_g[rb])

        def wait_gather(j, rb):
            pltpu.make_async_copy(h_hbm.at[srcs[j]], rows_b[rb],
                                  sem_g[rb]).wait()

        def issue_scatter(j, rb):
            pltpu.async_copy(rows_b[rb], acc.at[dsts[j]], sem_s[rb], add=True)

        def chunk_body(c, j, steady):
            rb = j % 2
            nrb = 1 - rb
            wait_gather(j, rb)
            if steady:
                wait_idx(c + 1, (j + 1) % 4)
                wait_scatter(nrb, (j + 3) % 4)
                issue_gather((j + 1) % 4, nrb)
                issue_idx(c + 2, (j + 2) % 4)
                compute_w((j + 1) % 4, nrb)
            else:
                @pl.when(c + 1 < NCHUNK)
                def _():
                    wait_idx(c + 1, (j + 1) % 4)

                @pl.when(c >= 1)
                def _():
                    wait_scatter(nrb, (j + 3) % 4)

                @pl.when(c + 1 < NCHUNK)
                def _():
                    issue_gather((j + 1) % 4, nrb)

                @pl.when(c + 2 < NCHUNK)
                def _():
                    issue_idx(c + 2, (j + 2) % 4)

                @pl.when(c + 1 < NCHUNK)
                def _():
                    compute_w((j + 1) % 4, nrb)
            scale_rows(rb)
            issue_scatter(j, rb)

        issue_idx(0, 0)
        wait_idx(0, 0)
        issue_idx(1, 1)
        issue_gather(0, 0)
        compute_w(0, 0)

        @pl.loop(0, 1)
        def _(q):
            for j in range(4):
                chunk_body(q * 4 + j, j, steady=False)

        @pl.loop(1, NCHUNK // 4)
        def _(q):
            for j in range(4):
                chunk_body(q * 4 + j, j, steady=True)

        chunk_body(NCHUNK - 1, 0, steady=False)

        wait_scatter(0, 0)

        plsc.subcore_barrier()

        @pl.when(sid < NS - 1)
        def _():
            pltpu.sync_copy(acc.at[pl.ds(row_start, RPT)],
                            acc_out.at[cid, pl.ds(row_start, RPT)])

        @pl.when(sid == NS - 1)
        def _():
            pltpu.sync_copy(acc.at[pl.ds(row_start, RPT_LAST)],
                            acc_out.at[cid, pl.ds(row_start, RPT_LAST)])

        pltpu.sync_copy(s_tile, s_out.at[wid, 0])

    return k


def _dense1_body(x_ref, W_ref, as_ref, ad_ref,
                 ha_ref, hb_ref, asn_ref, adn_ref, lw_ref):
    h = jnp.dot(x_ref[...], W_ref[...], preferred_element_type=jnp.float32)
    ha_ref[...] = h[:, :DO]
    hb_ref[...] = h[:, DO:]
    asn = jnp.sum(h * as_ref[...], axis=1)
    adn = jnp.sum(h * ad_ref[...], axis=1)
    asn_ref[...] = asn
    adn_ref[...] = adn
    e = asn + adn
    lw_ref[...] = jnp.exp(jnp.maximum(e, 0.2 * e))


def _combine2_body(accA_ref, accB_ref, sp_ref, ha_ref, hb_ref, lw_ref, b_ref,
                   W_ref, as_ref, ad_ref, h2_ref, asn_ref, adn_ref, lw2_ref):
    lw = lw_ref[...]
    s = jnp.sum(sp_ref[...][:, 0, :], axis=0) + lw
    inv = (1.0 / (s + 1e-16))[:, None]
    b = b_ref[...]
    oa = (accA_ref[0] + accA_ref[1] + lw[:, None] * ha_ref[...]) * inv + b[:, :DO]
    ob = (accB_ref[0] + accB_ref[1] + lw[:, None] * hb_ref[...]) * inv + b[:, DO:]
    oa = jnp.maximum(oa, 0.0)
    ob = jnp.maximum(ob, 0.0)
    W = W_ref[...]
    h2 = (jnp.dot(oa, W[:DO, :], preferred_element_type=jnp.float32)
          + jnp.dot(ob, W[DO:, :], preferred_element_type=jnp.float32))
    h2_ref[...] = h2
    asn = jnp.sum(h2 * as_ref[...], axis=1)
    adn = jnp.sum(h2 * ad_ref[...], axis=1)
    asn_ref[...] = asn
    adn_ref[...] = adn
    e2 = asn + adn
    lw2_ref[...] = jnp.exp(jnp.maximum(e2, 0.2 * e2))


def _final_body(acc_ref, sp_ref, h_ref, lw_ref, b_ref, out_ref):
    lw = lw_ref[...]
    acc = acc_ref[0] + acc_ref[1] + lw[:, None] * h_ref[...]
    s = jnp.sum(sp_ref[...][:, 0, :], axis=0) + lw
    o = acc / (s + 1e-16)[:, None] + b_ref[...]
    m = jnp.max(o, axis=1, keepdims=True)
    z = o - m
    out_ref[...] = z - jnp.log(jnp.sum(jnp.exp(z), axis=1, keepdims=True))


@functools.lru_cache(maxsize=1)
def _edge64():
    return _edge_pass(DO)


def kernel(x, edge_index, new_edge_indexs, W1, a_src1, a_dst1, b1,
           W2, a_src2, a_dst2, b2):
    f32 = jnp.float32
    src = edge_index[0]
    dst = edge_index[1]
    ep = _edge64()

    ha, hb, asn1, adn1, lw1 = pl.pallas_call(
        _dense1_body,
        out_shape=[
            jax.ShapeDtypeStruct((NN, DO), f32),
            jax.ShapeDtypeStruct((NN, DO), f32),
            jax.ShapeDtypeStruct((NN,), f32),
            jax.ShapeDtypeStruct((NN,), f32),
            jax.ShapeDtypeStruct((NN,), f32),
        ],
    )(x, W1, a_src1.reshape(1, -1), a_dst1.reshape(1, -1))

    accA, sA = ep(ha, asn1, adn1, src, dst)
    accB, _sB = ep(hb, asn1, adn1, src, dst)

    h2, asn2, adn2, lw2 = pl.pallas_call(
        _combine2_body,
        out_shape=[
            jax.ShapeDtypeStruct((NN, DO), f32),
            jax.ShapeDtypeStruct((NN,), f32),
            jax.ShapeDtypeStruct((NN,), f32),
            jax.ShapeDtypeStruct((NN,), f32),
        ],
    )(accA, accB, sA, ha, hb, lw1, b1.reshape(1, -1), W2,
      a_src2.reshape(1, -1), a_dst2.reshape(1, -1))

    acc2, s2 = ep(h2, asn2, adn2, src, dst)

    out = pl.pallas_call(
        _final_body,
        out_shape=jax.ShapeDtypeStruct((NN, DO), f32),
    )(acc2, s2, h2, lw2, b2.reshape(1, -1))
    return out
